# baseline (speedup 1.0000x reference)
import functools

import jax
import jax.numpy as jnp
from jax import lax
from jax.experimental import pallas as pl
from jax.experimental.pallas import tpu as pltpu
from jax.experimental.pallas import tpu_sc as plsc

NC = 2
NS = 16
NW = NC * NS
LANES = 16

CHUNK = 128
DEG_CHUNK = 2000


def _mesh():
    return plsc.VectorSubcoreMesh(
        core_axis_name="c", subcore_axis_name="s", num_cores=NC, num_subcores=NS
    )


_SC_PARAMS = pltpu.CompilerParams(needs_layout_passes=False)


def _degree_kernel(n_pad, n_edges):
    epw = n_edges // NW
    n_chunks = epw // DEG_CHUNK
    hist_len = 2 * n_pad

    @functools.partial(
        pl.kernel,
        mesh=_mesh(),
        out_type=jax.ShapeDtypeStruct((NW, hist_len), jnp.float32),
        scratch_types=[
            pltpu.VMEM((hist_len,), jnp.float32),
            pltpu.VMEM((DEG_CHUNK,), jnp.int32),
        ],
        compiler_params=_SC_PARAMS,
    )
    def deg_kernel(src_hbm, dst_hbm, out_hbm, hist, idxbuf):
        cid = lax.axis_index("c")
        sid = lax.axis_index("s")
        wid = sid * NC + cid

        zeros = jnp.zeros((LANES,), jnp.float32)
        ones = jnp.ones((LANES,), jnp.float32)

        def zero_body(i, _):
            hist[pl.ds(i * LANES, LANES)] = zeros
            return 0

        lax.fori_loop(0, hist_len // LANES, zero_body, 0)

        base_w = wid * epw

        def do_half(idx_hbm, col):
            def chunk_body(k, _):
                pltpu.sync_copy(idx_hbm.at[pl.ds(base_w + k * DEG_CHUNK, DEG_CHUNK)], idxbuf)

                def vec_body(j, _):
                    v = idxbuf[pl.ds(j * LANES, LANES)]
                    plsc.addupdate_scatter(hist, [v * 2 + col], ones)
                    return 0

                lax.fori_loop(0, DEG_CHUNK // LANES, vec_body, 0)
                return 0

            lax.fori_loop(0, n_chunks, chunk_body, 0)

        do_half(src_hbm, 0)
        do_half(dst_hbm, 1)
        pltpu.sync_copy(hist, out_hbm.at[wid])

    return deg_kernel


def _edge_kernel(n_pad, d, cpw):
    half = cpw // 2
    rows_per_tile = n_pad // NS
    n_copies = rows_per_tile // CHUNK

    @functools.partial(
        pl.kernel,
        mesh=_mesh(),
        out_type=jax.ShapeDtypeStruct((NC, n_pad, d), jnp.float32),
        scratch_types=[
            pltpu.VMEM_SHARED((n_pad, d), jnp.float32),
            pltpu.VMEM((half, CHUNK), jnp.int32),
            pltpu.VMEM((half, CHUNK), jnp.int32),
            pltpu.VMEM((CHUNK, d), jnp.float32),
            pltpu.VMEM((CHUNK, d), jnp.float32),
            pltpu.SemaphoreType.DMA,
            pltpu.SemaphoreType.DMA,
            pltpu.SemaphoreType.DMA,
            pltpu.SemaphoreType.DMA,
        ],
        compiler_params=_SC_PARAMS,
    )
    def edge_kernel(m_hbm, src_hbm, dst_hbm, out_hbm, agg, sidx, didx,
                    r0, r1, gs0, gs1, ss0, ss1):
        rows = (r0, r1)
        gsems = (gs0, gs1)
        ssems = (ss0, ss1)

        cid = lax.axis_index("c")
        sid = lax.axis_index("s")
        wid = sid * NC + cid

        zeros = jnp.zeros((LANES,), jnp.float32)

        def zrow(i, _):
            def zcol(jj, _):
                r0[i, pl.ds(jj * LANES, LANES)] = zeros
                return 0

            lax.fori_loop(0, d // LANES, zcol, 0)
            return 0

        lax.fori_loop(0, CHUNK, zrow, 0)

        row0 = sid * rows_per_tile
        for k in range(n_copies):
            pltpu.sync_copy(r0, agg.at[pl.ds(row0 + k * CHUNK, CHUNK)])
        plsc.subcore_barrier()

        def issue_gather(c, b):
            pltpu.async_copy(m_hbm.at[sidx.at[c]], rows[b], gsems[b])

        def wait_gather(c, b):
            pltpu.make_async_copy(m_hbm.at[sidx.at[c]], rows[b], gsems[b]).wait()

        def issue_scatter(c, b):
            pltpu.async_copy(rows[b], agg.at[didx.at[c]], ssems[b], add=True)

        def wait_scatter(c, b):
            pltpu.make_async_copy(rows[b], agg.at[didx.at[c]], ssems[b]).wait()

        for h in range(2):
            base = wid * cpw + h * half
            pltpu.sync_copy(src_hbm.at[pl.ds(base, half)], sidx)
            pltpu.sync_copy(dst_hbm.at[pl.ds(base, half)], didx)

            def body(c, _):
                pltpu.async_copy(m_hbm.at[sidx.at[c]], r0, gs0).wait()
                pltpu.sync_copy(r0, agg.at[didx.at[c]], add=True)
                return 0

            lax.fori_loop(0, half, body, 0)

        plsc.subcore_barrier()

        for k in range(n_copies):
            r = row0 + k * CHUNK
            pltpu.sync_copy(agg.at[pl.ds(r, CHUNK)], r0)
            pltpu.sync_copy(r0, out_hbm.at[cid, pl.ds(r, CHUNK)])

    return edge_kernel


def _norm_body(deg_ref, f_ref, norms_ref, m_ref):
    deg = jnp.sum(deg_ref[...], axis=0)
    norm = jnp.where(deg > 0, lax.rsqrt(deg), 0.0)
    norms_ref[...] = norm
    m_ref[...] = f_ref[...] * norm[:, 0:1]


def _norm_pass(deg_parts, features, n_nodes, d, row_block):
    grid = (n_nodes // row_block,)
    return pl.pallas_call(
        _norm_body,
        grid=grid,
        in_specs=[
            pl.BlockSpec((NW, row_block, 2), lambda i: (0, i, 0)),
            pl.BlockSpec((row_block, d), lambda i: (i, 0)),
        ],
        out_specs=[
            pl.BlockSpec((row_block, 2), lambda i: (i, 0)),
            pl.BlockSpec((row_block, d), lambda i: (i, 0)),
        ],
        out_shape=[
            jax.ShapeDtypeStruct((n_nodes, 2), jnp.float32),
            jax.ShapeDtypeStruct((n_nodes, d), jnp.float32),
        ],
    )(deg_parts, features)


def _layer_body(part_ref, norms_ref, w_ref, b_ref, out_ref, *, scale_out):
    agg = part_ref[0] + part_ref[1]
    norms = norms_ref[...]
    h = agg * norms[:, 1:2]
    y = jnp.dot(h, w_ref[...], preferred_element_type=jnp.float32) + b_ref[...]
    g = jax.nn.gelu(y)
    if scale_out:
        g = g * norms[:, 0:1]
    out_ref[...] = g


def _layer_pass(parts, norms, w, b, n_nodes, d, row_block, scale_out):
    grid = (n_nodes // row_block,)
    return pl.pallas_call(
        functools.partial(_layer_body, scale_out=scale_out),
        grid=grid,
        in_specs=[
            pl.BlockSpec((NC, row_block, d), lambda i: (0, i, 0)),
            pl.BlockSpec((row_block, 2), lambda i: (i, 0)),
            pl.BlockSpec((d, d), lambda i: (0, 0)),
            pl.BlockSpec((1, d), lambda i: (0, 0)),
        ],
        out_specs=pl.BlockSpec((row_block, d), lambda i: (i, 0)),
        out_shape=jax.ShapeDtypeStruct((n_nodes, d), jnp.float32),
    )(parts, norms, w, b.reshape(1, d))


@jax.jit
def kernel(features, edge_index, W1, b1, W2, b2, W3, b3):
    n_nodes, d = features.shape
    n_edges = edge_index.shape[1]
    n_pad = ((n_nodes + 1279) // 1280) * 1280
    f_pad = jnp.zeros((n_pad, d), features.dtype).at[:n_nodes].set(features)

    quantum = NW * CHUNK * 16
    e_pad = ((n_edges + quantum - 1) // quantum) * quantum
    cpw = e_pad // (NW * CHUNK)
    ei = jnp.full((2, e_pad), n_pad - 1, jnp.int32).at[:, :n_edges].set(edge_index)
    src = ei[0]
    dst = ei[1]
    src2 = src.reshape(e_pad // CHUNK, CHUNK)
    dst2 = dst.reshape(e_pad // CHUNK, CHUNK)

    deg_parts = _degree_kernel(n_pad, n_edges)(src, dst)
    deg_parts = deg_parts.reshape(NW, n_pad, 2)

    row_block = n_pad // 10
    norms, m = _norm_pass(deg_parts, f_pad, n_pad, d, row_block)

    edge_pass = _edge_kernel(n_pad, d, cpw)
    for w, b, last in ((W1, b1, False), (W2, b2, False), (W3, b3, True)):
        parts = edge_pass(m, src2, dst2)
        m = _layer_pass(parts, norms, w, b, n_pad, d, row_block,
                        scale_out=not last)
    return m[:n_nodes]


# speedup vs baseline: 1.4461x; 1.4461x over previous
import functools

import jax
import jax.numpy as jnp
from jax import lax
from jax.experimental import pallas as pl
from jax.experimental.pallas import tpu as pltpu
from jax.experimental.pallas import tpu_sc as plsc

NC = 2
NS = 16
NW = NC * NS
LANES = 16

CHUNK = 128
DEG_CHUNK = 2000


def _mesh():
    return plsc.VectorSubcoreMesh(
        core_axis_name="c", subcore_axis_name="s", num_cores=NC, num_subcores=NS
    )


_SC_PARAMS = pltpu.CompilerParams(needs_layout_passes=False)


def _degree_kernel(n_pad, n_edges):
    epw = n_edges // NW
    n_chunks = epw // DEG_CHUNK
    hist_len = 2 * n_pad

    @functools.partial(
        pl.kernel,
        mesh=_mesh(),
        out_type=jax.ShapeDtypeStruct((NW, hist_len), jnp.float32),
        scratch_types=[
            pltpu.VMEM((hist_len,), jnp.float32),
            pltpu.VMEM((DEG_CHUNK,), jnp.int32),
        ],
        compiler_params=_SC_PARAMS,
    )
    def deg_kernel(src_hbm, dst_hbm, out_hbm, hist, idxbuf):
        cid = lax.axis_index("c")
        sid = lax.axis_index("s")
        wid = sid * NC + cid

        zeros = jnp.zeros((LANES,), jnp.float32)
        ones = jnp.ones((LANES,), jnp.float32)

        def zero_body(i, _):
            hist[pl.ds(i * LANES, LANES)] = zeros
            return 0

        lax.fori_loop(0, hist_len // LANES, zero_body, 0)

        base_w = wid * epw

        def do_half(idx_hbm, col):
            def chunk_body(k, _):
                pltpu.sync_copy(idx_hbm.at[pl.ds(base_w + k * DEG_CHUNK, DEG_CHUNK)], idxbuf)

                def vec_body(j, _):
                    v = idxbuf[pl.ds(j * LANES, LANES)]
                    plsc.addupdate_scatter(hist, [v * 2 + col], ones)
                    return 0

                lax.fori_loop(0, DEG_CHUNK // LANES, vec_body, 0)
                return 0

            lax.fori_loop(0, n_chunks, chunk_body, 0)

        do_half(src_hbm, 0)
        do_half(dst_hbm, 1)
        pltpu.sync_copy(hist, out_hbm.at[wid])

    return deg_kernel


def _edge_kernel(n_pad, d, cpw):
    rows_per_tile = n_pad // NS
    n_copies = rows_per_tile // CHUNK

    @functools.partial(
        pl.kernel,
        mesh=_mesh(),
        out_type=jax.ShapeDtypeStruct((NC, n_pad, d), jnp.float32),
        scratch_types=[pltpu.VMEM_SHARED((n_pad, d), jnp.float32)]
        + [pltpu.VMEM((2, CHUNK), jnp.int32) for _ in range(4)]
        + [pltpu.VMEM((CHUNK, d), jnp.float32) for _ in range(2)]
        + [pltpu.SemaphoreType.DMA for _ in range(8)],
        compiler_params=_SC_PARAMS,
    )
    def edge_kernel(m_hbm, ei_hbm, out_hbm, agg,
                    i0, i1, i2, i3, r0, r1,
                    is0, is1, is2, is3, gs0, gs1, ss0, ss1):
        ibufs = (i0, i1, i2, i3)
        rows = (r0, r1)
        isems = (is0, is1, is2, is3)
        gsems = (gs0, gs1)
        ssems = (ss0, ss1)

        cid = lax.axis_index("c")
        sid = lax.axis_index("s")
        wid = sid * NC + cid

        zeros = jnp.zeros((LANES,), jnp.float32)

        def zrow(i, _):
            def zcol(jj, _):
                r0[i, pl.ds(jj * LANES, LANES)] = zeros
                return 0

            lax.fori_loop(0, d // LANES, zcol, 0)
            return 0

        lax.fori_loop(0, CHUNK, zrow, 0)

        row0 = sid * rows_per_tile
        for k in range(n_copies):
            pltpu.sync_copy(r0, agg.at[pl.ds(row0 + k * CHUNK, CHUNK)])
        plsc.subcore_barrier()

        base0 = wid * cpw * CHUNK

        def issue_idx(c, q):
            cc = jnp.minimum(c, cpw - 1)
            pltpu.async_copy(
                ei_hbm.at[:, pl.ds(base0 + cc * CHUNK, CHUNK)], ibufs[q], isems[q])

        def wait_idx(q):
            pltpu.make_async_copy(
                ei_hbm.at[:, pl.ds(base0, CHUNK)], ibufs[q], isems[q]).wait()

        def issue_gather(q, b):
            pltpu.async_copy(m_hbm.at[ibufs[q].at[0]], rows[b], gsems[b])

        def wait_gather(q, b):
            pltpu.make_async_copy(m_hbm.at[ibufs[q].at[0]], rows[b], gsems[b]).wait()

        def issue_scatter(q, b):
            pltpu.async_copy(rows[b], agg.at[ibufs[q].at[1]], ssems[b], add=True)

        def wait_scatter(q, b):
            pltpu.make_async_copy(rows[b], agg.at[ibufs[q].at[1]], ssems[b]).wait()

        issue_idx(0, 0)
        issue_idx(1, 1)
        issue_idx(2, 2)
        wait_idx(0)
        issue_gather(0, 0)
        issue_idx(3, 3)
        wait_idx(1)
        issue_gather(1, 1)
        wait_gather(0, 0)
        issue_scatter(0, 0)
        wait_scatter(0, 0)
        issue_idx(4, 0)
        wait_idx(2)
        issue_gather(2, 0)
        wait_gather(1, 1)
        issue_scatter(1, 1)
        wait_scatter(1, 1)
        issue_idx(5, 1)
        wait_idx(3)
        issue_gather(3, 1)
        wait_gather(2, 0)
        issue_scatter(2, 0)

        def body(j, _):
            for q in range(4):
                c = 4 * j + q
                b = q % 2
                wait_scatter((q + 2) % 4, b)
                issue_idx(c + 2, (q + 2) % 4)
                wait_idx(q)
                issue_gather(q, b)
                wait_gather((q + 3) % 4, 1 - b)
                issue_scatter((q + 3) % 4, 1 - b)
            return 0

        lax.fori_loop(1, cpw // 4, body, 0)

        wait_gather(3, 1)
        issue_scatter(3, 1)
        wait_scatter(2, 0)
        wait_scatter(3, 1)
        wait_idx(0)
        wait_idx(1)

        plsc.subcore_barrier()

        for k in range(n_copies):
            r = row0 + k * CHUNK
            pltpu.sync_copy(agg.at[pl.ds(r, CHUNK)], r0)
            pltpu.sync_copy(r0, out_hbm.at[cid, pl.ds(r, CHUNK)])

    return edge_kernel


def _norm_body(deg_ref, f_ref, norms_ref, m_ref):
    deg = jnp.sum(deg_ref[...], axis=0)
    norm = jnp.where(deg > 0, lax.rsqrt(deg), 0.0)
    norms_ref[...] = norm
    m_ref[...] = f_ref[...] * norm[:, 0:1]


def _norm_pass(deg_parts, features, n_nodes, d, row_block):
    grid = (n_nodes // row_block,)
    return pl.pallas_call(
        _norm_body,
        grid=grid,
        in_specs=[
            pl.BlockSpec((NW, row_block, 2), lambda i: (0, i, 0)),
            pl.BlockSpec((row_block, d), lambda i: (i, 0)),
        ],
        out_specs=[
            pl.BlockSpec((row_block, 2), lambda i: (i, 0)),
            pl.BlockSpec((row_block, d), lambda i: (i, 0)),
        ],
        out_shape=[
            jax.ShapeDtypeStruct((n_nodes, 2), jnp.float32),
            jax.ShapeDtypeStruct((n_nodes, d), jnp.float32),
        ],
    )(deg_parts, features)


def _layer_body(part_ref, norms_ref, w_ref, b_ref, out_ref, *, scale_out):
    agg = part_ref[0] + part_ref[1]
    norms = norms_ref[...]
    h = agg * norms[:, 1:2]
    y = jnp.dot(h, w_ref[...], preferred_element_type=jnp.float32) + b_ref[...]
    g = jax.nn.gelu(y)
    if scale_out:
        g = g * norms[:, 0:1]
    out_ref[...] = g


def _layer_pass(parts, norms, w, b, n_nodes, d, row_block, scale_out):
    grid = (n_nodes // row_block,)
    return pl.pallas_call(
        functools.partial(_layer_body, scale_out=scale_out),
        grid=grid,
        in_specs=[
            pl.BlockSpec((NC, row_block, d), lambda i: (0, i, 0)),
            pl.BlockSpec((row_block, 2), lambda i: (i, 0)),
            pl.BlockSpec((d, d), lambda i: (0, 0)),
            pl.BlockSpec((1, d), lambda i: (0, 0)),
        ],
        out_specs=pl.BlockSpec((row_block, d), lambda i: (i, 0)),
        out_shape=jax.ShapeDtypeStruct((n_nodes, d), jnp.float32),
    )(parts, norms, w, b.reshape(1, d))


@jax.jit
def kernel(features, edge_index, W1, b1, W2, b2, W3, b3):
    n_nodes, d = features.shape
    n_edges = edge_index.shape[1]
    n_pad = ((n_nodes + 1279) // 1280) * 1280
    f_pad = jnp.zeros((n_pad, d), features.dtype).at[:n_nodes].set(features)

    quantum = NW * CHUNK * 4
    e_pad = ((n_edges + quantum - 1) // quantum) * quantum
    cpw = e_pad // (NW * CHUNK)
    ei = jnp.full((2, e_pad), n_pad - 1, jnp.int32).at[:, :n_edges].set(edge_index)
    src = ei[0]
    dst = ei[1]

    deg_parts = _degree_kernel(n_pad, n_edges)(src, dst)
    deg_parts = deg_parts.reshape(NW, n_pad, 2)

    row_block = n_pad // 10
    norms, m = _norm_pass(deg_parts, f_pad, n_pad, d, row_block)

    edge_pass = _edge_kernel(n_pad, d, cpw)
    for w, b, last in ((W1, b1, False), (W2, b2, False), (W3, b3, True)):
        parts = edge_pass(m, ei)
        m = _layer_pass(parts, norms, w, b, n_pad, d, row_block,
                        scale_out=not last)
    return m[:n_nodes]
